# Initial kernel scaffold; baseline (speedup 1.0000x reference)
import jax
import jax.numpy as jnp
from jax.experimental import pallas as pl


def kernel(x, digit_emb, row_emb, col_emb):
    raise NotImplementedError("write your pallas kernel here")


# speedup vs baseline: 1.4288x; 1.4288x over previous
import functools

import jax
import jax.numpy as jnp
from jax import lax
from jax.experimental import pallas as pl
from jax.experimental.pallas import tpu as pltpu
from jax.experimental.pallas import tpu_sc as plsc

DIGIT_DIM = 16
POS_DIM = 8
OUT_DIM = DIGIT_DIM + 2 * POS_DIM

BATCH = 16384
NPOS = 81
NTOK = BATCH * NPOS
NWORKERS = 32
IDX_ROW = 128
N_IDX_ROWS = NTOK // IDX_ROW
ROWS_PER_TILE = N_IDX_ROWS // NWORKERS
GPB = 6
ITERS = ROWS_PER_TILE // GPB
CHUNK = GPB * IDX_ROW


def _sc_gather(tab, idx2d):
    mesh = plsc.VectorSubcoreMesh(core_axis_name="c", subcore_axis_name="s")

    @functools.partial(
        pl.kernel,
        mesh=mesh,
        compiler_params=pltpu.CompilerParams(use_tc_tiling_on_sc=False),
        out_type=jax.ShapeDtypeStruct((NTOK, OUT_DIM), jnp.float32),
        scratch_types=[
            pltpu.VMEM((ROWS_PER_TILE, IDX_ROW), jnp.int32),
            pltpu.VMEM((CHUNK, OUT_DIM), jnp.float32),
            pltpu.SemaphoreType.DMA,
        ],
    )
    def body(tab_hbm, idx_hbm, out_hbm, idx_v, buf_v, sem):
        wid = lax.axis_index("s") * 2 + lax.axis_index("c")
        pltpu.sync_copy(idx_hbm.at[wid], idx_v)
        out_base = wid * (ROWS_PER_TILE * IDX_ROW)

        def step(i, carry):
            cps = [
                pltpu.async_copy(
                    tab_hbm.at[idx_v.at[i * GPB + g]],
                    buf_v.at[pl.ds(g * IDX_ROW, IDX_ROW)],
                    sem)
                for g in range(GPB)
            ]
            for cp in cps:
                cp.wait()
            pltpu.sync_copy(
                buf_v, out_hbm.at[pl.ds(out_base + i * CHUNK, CHUNK)])
            return carry

        lax.fori_loop(0, ITERS, step, 0)

    return body(tab, idx2d)


def kernel(x, digit_emb, row_emb, col_emb):
    pos = jnp.arange(NPOS, dtype=jnp.int32)
    posemb = jnp.concatenate(
        [jnp.take(row_emb, pos // 9, axis=0),
         jnp.take(col_emb, pos % 9, axis=0)], axis=-1)
    tab = jnp.concatenate(
        [jnp.broadcast_to(digit_emb[None, :, :], (NPOS, 10, DIGIT_DIM)),
         jnp.broadcast_to(posemb[:, None, :], (NPOS, 10, 2 * POS_DIM))],
        axis=-1).reshape(NPOS * 10, OUT_DIM)
    idx = (x.astype(jnp.int32) + pos[None, :] * 10).reshape(
        NWORKERS, ROWS_PER_TILE, IDX_ROW)
    out = _sc_gather(tab, idx)
    return out.reshape(BATCH, NPOS, OUT_DIM)


# speedup vs baseline: 6.5306x; 4.5706x over previous
import functools

import jax
import jax.numpy as jnp
from jax import lax
from jax.experimental import pallas as pl
from jax.experimental.pallas import tpu as pltpu
from jax.experimental.pallas import tpu_sc as plsc

DIGIT_DIM = 16
POS_DIM = 8
OUT_DIM = DIGIT_DIM + 2 * POS_DIM

BATCH = 16384
NPOS = 81
NWORKERS = 32
BOARDS_PER_TILE = BATCH // NWORKERS
GPB = 16
ITERS = BOARDS_PER_TILE // GPB


def _sc_gather(tab, idx):
    mesh = plsc.VectorSubcoreMesh(core_axis_name="c", subcore_axis_name="s")

    @functools.partial(
        pl.kernel,
        mesh=mesh,
        compiler_params=pltpu.CompilerParams(use_tc_tiling_on_sc=False),
        out_type=jax.ShapeDtypeStruct((BATCH, NPOS, OUT_DIM), jnp.float32),
        scratch_types=[
            pltpu.VMEM((BOARDS_PER_TILE, NPOS), jnp.int32),
            pltpu.VMEM((GPB, NPOS, OUT_DIM), jnp.float32),
            pltpu.SemaphoreType.DMA,
        ],
    )
    def body(tab_hbm, idx_hbm, out_hbm, idx_v, buf_v, sem):
        wid = lax.axis_index("s") * 2 + lax.axis_index("c")
        b0 = wid * BOARDS_PER_TILE
        pltpu.sync_copy(idx_hbm.at[pl.ds(b0, BOARDS_PER_TILE)], idx_v)

        def step(i, carry):
            cps = [
                pltpu.async_copy(
                    tab_hbm.at[idx_v.at[i * GPB + j]], buf_v.at[j], sem)
                for j in range(GPB)
            ]
            for cp in cps:
                cp.wait()
            pltpu.sync_copy(buf_v, out_hbm.at[pl.ds(b0 + i * GPB, GPB)])
            return carry

        lax.fori_loop(0, ITERS, step, 0)

    return body(tab, idx)


def kernel(x, digit_emb, row_emb, col_emb):
    pos = jnp.arange(NPOS, dtype=jnp.int32)
    posemb = jnp.concatenate(
        [jnp.take(row_emb, pos // 9, axis=0),
         jnp.take(col_emb, pos % 9, axis=0)], axis=-1)
    tab = jnp.concatenate(
        [jnp.broadcast_to(digit_emb[None, :, :], (NPOS, 10, DIGIT_DIM)),
         jnp.broadcast_to(posemb[:, None, :], (NPOS, 10, 2 * POS_DIM))],
        axis=-1).reshape(NPOS * 10, OUT_DIM)
    idx = x.astype(jnp.int32) + pos[None, :] * 10
    return _sc_gather(tab, idx)
